# baseline (speedup 1.0000x reference)
import functools

import jax
import jax.numpy as jnp
import numpy as np
from jax import lax
from jax.experimental import pallas as pl
from jax.experimental.pallas import tpu as pltpu
from jax.experimental.pallas import tpu_sc as plsc

BATCH = 16384
NUM_INPUTS = 2048
TUPLE_SIZE = 8
NUM_LUTS = NUM_INPUTS // TUPLE_SIZE
LUT_ENTRIES = 1 << TUPLE_SIZE
PAIRS = NUM_LUTS // 2


NCHUNK = 1
BCHUNK = BATCH // NCHUNK
TC_ROWS = 2048


def _pack_body(x_ref, pt_ref, out_ref):
    xb = x_ref[...].astype(jnp.bfloat16)
    a = lax.dot_general(pt_ref[...], xb, (((1,), (1,)), ((), ())),
                        preferred_element_type=jnp.float32)
    ai = a.astype(jnp.int32)
    roff = lax.broadcasted_iota(jnp.int32, (PAIRS, TC_ROWS), 0) * (2 * LUT_ENTRIES)
    f_e = ai[:PAIRS] + roff
    f_o = ai[PAIRS:] + (roff + LUT_ENTRIES)
    out_ref[...] = f_e | (f_o << 16)


def _pack_addresses(x, pt, c):
    blk0 = c * (BCHUNK // TC_ROWS)
    return pl.pallas_call(
        _pack_body,
        grid=(BCHUNK // TC_ROWS,),
        in_specs=[
            pl.BlockSpec((TC_ROWS, NUM_INPUTS), lambda i: (blk0 + i, 0)),
            pl.BlockSpec((NUM_LUTS, NUM_INPUTS), lambda i: (0, 0)),
        ],
        out_specs=pl.BlockSpec((PAIRS, TC_ROWS), lambda i: (0, i)),
        out_shape=jax.ShapeDtypeStruct((PAIRS, BCHUNK), jnp.int32),
    )(x, pt)


NW = 32
COLS_PER_TILE = BCHUNK // NW
CHUNK = 128
LANES = 16
GROUPS = CHUNK // LANES
TABLE = NUM_LUTS * LUT_ENTRIES
MASK16 = (1 << 16) - 1


def _sc_body(fidx_hbm, lut_hbm, out_hbm, lut_v, idx_v, out_v):
    wid = lax.axis_index("s") * 2 + lax.axis_index("c")
    base_col = wid * COLS_PER_TILE
    pltpu.sync_copy(lut_hbm, lut_v)

    def chunk_body(ci, _):
        col0 = base_col + ci * CHUNK
        pltpu.sync_copy(fidx_hbm.at[:, pl.ds(col0, CHUNK)], idx_v)

        def t_body(t, accs):
            new = []
            for g in range(GROUPS):
                v = idx_v[t, pl.ds(g * LANES, LANES)]
                lo = v & MASK16
                hi = lax.shift_right_logical(v, 16)
                acc = accs[g] + plsc.load_gather(lut_v, [lo])
                new.append(acc + plsc.load_gather(lut_v, [hi]))
            return tuple(new)

        zeros = jnp.zeros((LANES,), jnp.float32)
        accs = lax.fori_loop(0, PAIRS, t_body, (zeros,) * GROUPS)
        for g in range(GROUPS):
            out_v[pl.ds(ci * CHUNK + g * LANES, LANES)] = accs[g]
        return 0

    lax.fori_loop(0, COLS_PER_TILE // CHUNK, chunk_body, 0)
    pltpu.sync_copy(out_v, out_hbm.at[pl.ds(base_col, COLS_PER_TILE)])


_sc_gather = functools.partial(
    pl.kernel,
    out_type=jax.ShapeDtypeStruct((BCHUNK,), jnp.float32),
    mesh=plsc.VectorSubcoreMesh(core_axis_name="c", subcore_axis_name="s"),
    compiler_params=pltpu.CompilerParams(needs_layout_passes=False),
    scratch_types=[
        pltpu.VMEM((TABLE,), jnp.float32),
        pltpu.VMEM((PAIRS, CHUNK), jnp.int32),
        pltpu.VMEM((COLS_PER_TILE,), jnp.float32),
    ],
)(_sc_body)


def _build_pt():
    k = np.arange(NUM_INPUTS)
    grp = k[None, :] // TUPLE_SIZE
    pw = (1 << (k % TUPLE_SIZE))[None, :]
    rows = np.concatenate([2 * np.arange(PAIRS), 2 * np.arange(PAIRS) + 1])
    return np.asarray((grp == rows[:, None]) * pw).astype(jnp.bfloat16)


_PT = _build_pt()


def kernel(x, lut_weights):
    lut_flat = lut_weights.reshape(-1)
    outs = []
    for c in range(NCHUNK):
        fidx = _pack_addresses(x, _PT, c)
        outs.append(_sc_gather(fidx, lut_flat))
    out = outs[0] if NCHUNK == 1 else jnp.concatenate(outs)
    return out


# speedup vs baseline: 1.0703x; 1.0703x over previous
import functools

import jax
import jax.numpy as jnp
import numpy as np
from jax import lax
from jax.experimental import pallas as pl
from jax.experimental.pallas import tpu as pltpu
from jax.experimental.pallas import tpu_sc as plsc

BATCH = 16384
NUM_INPUTS = 2048
TUPLE_SIZE = 8
NUM_LUTS = NUM_INPUTS // TUPLE_SIZE
LUT_ENTRIES = 1 << TUPLE_SIZE
PAIRS = NUM_LUTS // 2


NCHUNK = 1
BCHUNK = BATCH // NCHUNK
TC_ROWS = 1024


def _pack_body(x_ref, pt_ref, out_ref):
    xb = x_ref[...].astype(jnp.bfloat16)
    a = lax.dot_general(pt_ref[...], xb, (((1,), (1,)), ((), ())),
                        preferred_element_type=jnp.float32)
    ai = a.astype(jnp.int32)
    roff = lax.broadcasted_iota(jnp.int32, (PAIRS, TC_ROWS), 0) * (2 * LUT_ENTRIES)
    f_e = ai[:PAIRS] + roff
    f_o = ai[PAIRS:] + (roff + LUT_ENTRIES)
    out_ref[...] = f_e | (f_o << 16)


def _pack_addresses(x, pt, c):
    blk0 = c * (BCHUNK // TC_ROWS)
    return pl.pallas_call(
        _pack_body,
        grid=(BCHUNK // TC_ROWS,),
        in_specs=[
            pl.BlockSpec((TC_ROWS, NUM_INPUTS), lambda i: (blk0 + i, 0)),
            pl.BlockSpec((NUM_LUTS, NUM_INPUTS), lambda i: (0, 0)),
        ],
        out_specs=pl.BlockSpec((PAIRS, TC_ROWS), lambda i: (0, i)),
        out_shape=jax.ShapeDtypeStruct((PAIRS, BCHUNK), jnp.int32),
    )(x, pt)


NW = 32
COLS_PER_TILE = BCHUNK // NW
CHUNK = 128
LANES = 16
GROUPS = CHUNK // LANES
TABLE = NUM_LUTS * LUT_ENTRIES
MASK16 = (1 << 16) - 1


NCHUNKS_SC = COLS_PER_TILE // CHUNK


def _sc_body(fidx_hbm, lut_hbm, out_hbm, lut_v, idx_v0, idx_v1, out_v,
             sem0, sem1):
    wid = lax.axis_index("s") * 2 + lax.axis_index("c")
    base_col = wid * COLS_PER_TILE
    bufs = (idx_v0, idx_v1)
    sems = (sem0, sem1)

    handles = {}
    for ci in range(min(2, NCHUNKS_SC)):
        col0 = base_col + ci * CHUNK
        handles[ci] = pltpu.async_copy(
            fidx_hbm.at[:, pl.ds(col0, CHUNK)], bufs[ci % 2], sems[ci % 2]
        )
    pltpu.sync_copy(lut_hbm, lut_v)

    for ci in range(NCHUNKS_SC):
        handles[ci].wait()
        buf = bufs[ci % 2]

        def t_body(t, accs, buf=buf):
            new = []
            for g in range(GROUPS):
                v = buf[t, pl.ds(g * LANES, LANES)]
                lo = v & MASK16
                hi = lax.shift_right_logical(v, 16)
                acc = accs[g] + plsc.load_gather(lut_v, [lo])
                new.append(acc + plsc.load_gather(lut_v, [hi]))
            return tuple(new)

        zeros = jnp.zeros((LANES,), jnp.float32)
        accs = lax.fori_loop(0, PAIRS, t_body, (zeros,) * GROUPS)
        for g in range(GROUPS):
            out_v[pl.ds(ci * CHUNK + g * LANES, LANES)] = accs[g]
        if ci + 2 < NCHUNKS_SC:
            col0 = base_col + (ci + 2) * CHUNK
            handles[ci + 2] = pltpu.async_copy(
                fidx_hbm.at[:, pl.ds(col0, CHUNK)], bufs[ci % 2], sems[ci % 2]
            )

    pltpu.sync_copy(out_v, out_hbm.at[pl.ds(base_col, COLS_PER_TILE)])


_sc_gather = functools.partial(
    pl.kernel,
    out_type=jax.ShapeDtypeStruct((BCHUNK,), jnp.float32),
    mesh=plsc.VectorSubcoreMesh(core_axis_name="c", subcore_axis_name="s"),
    compiler_params=pltpu.CompilerParams(needs_layout_passes=False),
    scratch_types=[
        pltpu.VMEM((TABLE,), jnp.float32),
        pltpu.VMEM((PAIRS, CHUNK), jnp.int32),
        pltpu.VMEM((PAIRS, CHUNK), jnp.int32),
        pltpu.VMEM((COLS_PER_TILE,), jnp.float32),
        pltpu.SemaphoreType.DMA,
        pltpu.SemaphoreType.DMA,
    ],
)(_sc_body)


def _build_pt():
    k = np.arange(NUM_INPUTS)
    grp = k[None, :] // TUPLE_SIZE
    pw = (1 << (k % TUPLE_SIZE))[None, :]
    rows = np.concatenate([2 * np.arange(PAIRS), 2 * np.arange(PAIRS) + 1])
    return np.asarray((grp == rows[:, None]) * pw).astype(jnp.bfloat16)


_PT = _build_pt()


def kernel(x, lut_weights):
    lut_flat = lut_weights.reshape(-1)
    outs = []
    for c in range(NCHUNK):
        fidx = _pack_addresses(x, _PT, c)
        outs.append(_sc_gather(fidx, lut_flat))
    out = outs[0] if NCHUNK == 1 else jnp.concatenate(outs)
    return out
